# Initial kernel scaffold; baseline (speedup 1.0000x reference)
import jax
import jax.numpy as jnp
from jax.experimental import pallas as pl


def kernel(embeddings, target):
    raise NotImplementedError("write your pallas kernel here")


# speedup vs baseline: 1.0256x; 1.0256x over previous
import jax
import jax.numpy as jnp
from jax.experimental import pallas as pl
from jax.experimental.pallas import tpu as pltpu

MARGIN_ = 1.0
BIG_ = 1e9
TILE_ = 512


def _triplet_kernel(rows_ref, emb_ref, tgt_ref, loss_ref, cnt_ref):
    i = pl.program_id(0)
    n_steps = pl.num_programs(0)

    emb_all = emb_ref[...]
    emb_r = rows_ref[...]
    t_all = tgt_ref[0, :]
    t_r = tgt_ref[0, pl.ds(i * TILE_, TILE_)]

    sq_all = jnp.sum(emb_all * emb_all, axis=1)
    sq_r = jnp.sum(emb_r * emb_r, axis=1)

    dot = jnp.dot(emb_r, emb_all.T, preferred_element_type=jnp.float32)
    D = jnp.maximum(sq_r[:, None] + sq_all[None, :] - 2.0 * dot, 0.0)

    B = emb_all.shape[0]
    col_ids = jax.lax.broadcasted_iota(jnp.int32, (TILE_, B), 1)
    row_ids = jax.lax.broadcasted_iota(jnp.int32, (TILE_, B), 0) + i * TILE_
    same = t_r[:, None] == t_all[None, :]
    diag = row_ids == col_ids
    pos_mask = same & (~diag)
    neg_mask = ~same

    ap = jnp.max(jnp.where(pos_mask, D, -BIG_), axis=1)
    an = jnp.min(jnp.where(neg_mask, D, BIG_), axis=1)
    has_pos = jnp.max(pos_mask.astype(jnp.int32), axis=1) > 0
    has_neg = jnp.max(neg_mask.astype(jnp.int32), axis=1) > 0
    valid = has_pos & has_neg

    losses = jnp.maximum(ap - an + MARGIN_, 0.0)
    part_loss = jnp.sum(jnp.where(valid, losses, 0.0))
    part_cnt = jnp.sum(valid.astype(jnp.int32))

    @pl.when(i == 0)
    def _init():
        loss_ref[...] = part_loss.reshape(1, 1)
        cnt_ref[...] = part_cnt.reshape(1, 1)

    @pl.when(i > 0)
    def _acc():
        loss_ref[...] = loss_ref[...] + part_loss.reshape(1, 1)
        cnt_ref[...] = cnt_ref[...] + part_cnt.reshape(1, 1)

    @pl.when(i == n_steps - 1)
    def _final():
        denom = jnp.maximum(cnt_ref[...], 1).astype(jnp.float32)
        loss_ref[...] = loss_ref[...] / denom


def kernel(embeddings, target):
    B = embeddings.shape[0]
    tgt2d = target.reshape(1, B)
    n_steps = B // TILE_

    loss, cnt = pl.pallas_call(
        _triplet_kernel,
        grid=(n_steps,),
        in_specs=[
            pl.BlockSpec((TILE_, embeddings.shape[1]), lambda i: (i, 0)),
            pl.BlockSpec((B, embeddings.shape[1]), lambda i: (0, 0)),
            pl.BlockSpec((1, B), lambda i: (0, 0)),
        ],
        out_specs=[
            pl.BlockSpec((1, 1), lambda i: (0, 0)),
            pl.BlockSpec((1, 1), lambda i: (0, 0)),
        ],
        out_shape=[
            jax.ShapeDtypeStruct((1, 1), jnp.float32),
            jax.ShapeDtypeStruct((1, 1), jnp.int32),
        ],
    )(embeddings, embeddings, tgt2d)
    return (loss[0, 0], cnt[0, 0])


# speedup vs baseline: 1.9336x; 1.8854x over previous
import jax
import jax.numpy as jnp
from jax.experimental import pallas as pl

MARGIN_ = 1.0
BIG_ = 1e9
TILE_ = 512


def _triplet_kernel(rows_ref, emb_ref, tgt_ref, loss_ref, cnt_ref):
    i = pl.program_id(0)
    n_steps = pl.num_programs(0)

    emb_all = emb_ref[...]
    emb_r = rows_ref[...]
    t_all = tgt_ref[0, :]
    t_r = tgt_ref[0, pl.ds(i * TILE_, TILE_)]
    B = emb_all.shape[0]

    sq_all = jnp.sum(emb_all * emb_all, axis=1)
    sq_r = jnp.sum(emb_r * emb_r, axis=1)

    ones_r = jnp.ones((TILE_, 1), dtype=jnp.float32)
    ones_c = jnp.ones((B, 1), dtype=jnp.float32)
    a_aug = jnp.concatenate([emb_r, ones_r, sq_r[:, None]], axis=1)
    b_aug = jnp.concatenate([emb_all * -2.0, sq_all[:, None], ones_c], axis=1)
    D = jnp.dot(a_aug, b_aug.T, preferred_element_type=jnp.float32)

    same = t_r[:, None] == t_all[None, :]
    ap = jnp.maximum(jnp.max(jnp.where(same, D, -BIG_), axis=1), 0.0)
    an = jnp.maximum(jnp.min(jnp.where(same, BIG_, D), axis=1), 0.0)
    cnt_same = jnp.sum(same.astype(jnp.int32), axis=1)
    valid = (cnt_same >= 2) & (cnt_same < B)

    losses = jnp.maximum(ap - an + MARGIN_, 0.0)
    part_loss = jnp.sum(jnp.where(valid, losses, 0.0))
    part_cnt = jnp.sum(valid.astype(jnp.int32))

    @pl.when(i == 0)
    def _init():
        loss_ref[...] = part_loss.reshape(1, 1)
        cnt_ref[...] = part_cnt.reshape(1, 1)

    @pl.when(i > 0)
    def _acc():
        loss_ref[...] = loss_ref[...] + part_loss.reshape(1, 1)
        cnt_ref[...] = cnt_ref[...] + part_cnt.reshape(1, 1)

    @pl.when(i == n_steps - 1)
    def _final():
        denom = jnp.maximum(cnt_ref[...], 1).astype(jnp.float32)
        loss_ref[...] = loss_ref[...] / denom


def kernel(embeddings, target):
    B = embeddings.shape[0]
    tgt2d = target.reshape(1, B)
    n_steps = B // TILE_

    loss, cnt = pl.pallas_call(
        _triplet_kernel,
        grid=(n_steps,),
        in_specs=[
            pl.BlockSpec((TILE_, embeddings.shape[1]), lambda i: (i, 0)),
            pl.BlockSpec((B, embeddings.shape[1]), lambda i: (0, 0)),
            pl.BlockSpec((1, B), lambda i: (0, 0)),
        ],
        out_specs=[
            pl.BlockSpec((1, 1), lambda i: (0, 0)),
            pl.BlockSpec((1, 1), lambda i: (0, 0)),
        ],
        out_shape=[
            jax.ShapeDtypeStruct((1, 1), jnp.float32),
            jax.ShapeDtypeStruct((1, 1), jnp.int32),
        ],
    )(embeddings, embeddings, tgt2d)
    return (loss[0, 0], cnt[0, 0])
